# baseline (speedup 1.0000x reference)
import functools

import jax
import jax.numpy as jnp
from jax import lax
from jax.experimental import pallas as pl
from jax.experimental.pallas import tpu as pltpu
from jax.experimental.pallas import tpu_sc as plsc

N = 4096
D = 1024
BLK = 512
NBLK = N // BLK

NWORK = 32
RPW = N // NWORK


def _lin_body(x_ref, wg_ref, bg_ref, out_ref):
    out_ref[...] = (
        lax.dot_general(
            wg_ref[...], x_ref[...], (((1,), (1,)), ((), ())),
            preferred_element_type=jnp.float32,
        )
        + bg_ref[...]
    )


_sc_mesh = plsc.VectorSubcoreMesh(core_axis_name="c", subcore_axis_name="s")


@functools.partial(
    pl.kernel,
    mesh=_sc_mesh,
    out_type=jax.ShapeDtypeStruct((2 * N,), jnp.float32),
    scratch_types=[
        pltpu.VMEM((RPW + 8,), jnp.float32),
        pltpu.VMEM((RPW + 8,), jnp.float32),
        pltpu.VMEM((RPW,), jnp.float32),
        pltpu.VMEM((RPW,), jnp.float32),
    ],
)
def _sc_combine(v_hbm, out_hbm, va, vb, oa, ob):
    wid = lax.axis_index("s") * 2 + lax.axis_index("c")
    base = wid * RPW
    wrap = lax.rem(base + RPW, N)
    pltpu.sync_copy(v_hbm.at[pl.ds(base, RPW)], va.at[pl.ds(0, RPW)])
    pltpu.sync_copy(v_hbm.at[pl.ds(wrap, 8)], va.at[pl.ds(RPW, 8)])
    pltpu.sync_copy(v_hbm.at[pl.ds(N + base, RPW)], vb.at[pl.ds(0, RPW)])
    pltpu.sync_copy(v_hbm.at[pl.ds(N + wrap, 8)], vb.at[pl.ds(RPW, 8)])

    for j in range(RPW // 16):
        o = j * 16
        f0 = va[pl.ds(o, 16)]
        f1 = vb[pl.ds(o, 16)]
        a1 = va[pl.ds(o + 1, 16)]
        a2 = va[pl.ds(o + 2, 16)]
        b1 = vb[pl.ds(o + 1, 16)]
        b2 = vb[pl.ds(o + 2, 16)]
        oa[pl.ds(o, 16)] = f0 * a1 + f1 * a2
        ob[pl.ds(o, 16)] = f0 * b1 + f1 * b2

    pltpu.sync_copy(oa, out_hbm.at[pl.ds(base, RPW)])
    pltpu.sync_copy(ob, out_hbm.at[pl.ds(N + base, RPW)])


def kernel(X, Wf, bf, Wg, bg):
    del Wf, bf
    bg2 = bg.reshape(2, 1)
    Vt = pl.pallas_call(
        _lin_body,
        grid=(NBLK,),
        in_specs=[
            pl.BlockSpec((BLK, D), lambda i: (i, 0)),
            pl.BlockSpec((2, D), lambda i: (0, 0)),
            pl.BlockSpec((2, 1), lambda i: (0, 0)),
        ],
        out_specs=pl.BlockSpec((2, BLK), lambda i: (0, i)),
        out_shape=jax.ShapeDtypeStruct((2, N), jnp.float32),
    )(X, Wg, bg2)
    out_flat = _sc_combine(Vt.reshape(2 * N))
    return out_flat.reshape(2, N).T
